# baseline (device time: 157817 ns/iter reference)
import jax
import jax.numpy as jnp
from jax import lax
from jax.experimental import pallas as pl
from jax.experimental.pallas import tpu as pltpu

N_DEV = 4
SUBS = 2


def kernel(x, w_mat):
    m, k = x.shape
    _, n = w_mat.shape
    chunk = m // N_DEV
    half = n // 2
    subrows = chunk // SUBS
    n_slots = 6 * SUBS
    n_dmas = 2 * SUBS * N_DEV

    def body(x_ref, w_ref, out_ref, acc_ref, xv_ref, wv_ref, cw_ref, ccw_ref,
             cw_send, cw_recv, ccw_send, ccw_recv, out_sems, in_sems):
        d = lax.axis_index("i")
        right = lax.rem(d + 1, N_DEV)
        left = lax.rem(d + 3, N_DEV)

        cw_cols = pl.ds(0, half)
        ccw_cols = pl.ds(half, half)

        def srows(c, j):
            return pl.ds(c * chunk + j * subrows, subrows)

        def crows(c):
            return pl.ds(c * chunk, chunk)

        def slot(step, j):
            return step * SUBS + j

        def mk(step, j, src_cw, src_ccw, cw_tgt=None, ccw_tgt=None):
            s_ = slot(step, j)
            cw = pltpu.make_async_remote_copy(
                src_ref=src_cw, dst_ref=cw_ref.at[s_],
                send_sem=cw_send.at[s_], recv_sem=cw_recv.at[s_],
                device_id=(right if cw_tgt is None else cw_tgt,),
                device_id_type=pl.DeviceIdType.MESH)
            ccw = pltpu.make_async_remote_copy(
                src_ref=src_ccw, dst_ref=ccw_ref.at[s_],
                send_sem=ccw_send.at[s_], recv_sem=ccw_recv.at[s_],
                device_id=(left if ccw_tgt is None else ccw_tgt,),
                device_id_type=pl.DeviceIdType.MESH)
            return cw, ccw

        def start(pair):
            pair[0].start()
            pair[1].start()

        def wait(pair):
            pair[0].wait()
            pair[1].wait()

        out_dmas = []

        def dma_out(src, dst):
            cp = pltpu.make_async_copy(src, dst, out_sems.at[len(out_dmas)])
            cp.start()
            out_dmas.append(cp)

        def gemm(rs, cols, wcols):
            acc_ref[rs, cols] = jnp.dot(
                xv_ref[rs, :], wv_ref[:, wcols],
                preferred_element_type=jnp.float32)

        c_dm1 = lax.rem(d + 3, N_DEV)
        c_dp1 = lax.rem(d + 1, N_DEV)
        c_dp2 = lax.rem(d + 2, N_DEV)
        oc_cw = c_dp1
        oc_ccw = c_dm1

        w_cp = pltpu.make_async_copy(w_ref, wv_ref, in_sems.at[0])
        w_cp.start()
        x_cps = []
        for i, c_off in enumerate((3, 1, 2, 0)):
            c = lax.rem(d + c_off, N_DEV)
            cp = pltpu.make_async_copy(x_ref.at[crows(c), :],
                                       xv_ref.at[crows(c), :],
                                       in_sems.at[1 + i])
            cp.start()
            x_cps.append(cp)

        w_cp.wait()
        x_cps[0].wait()
        gemm(srows(c_dm1, 0), cw_cols, cw_cols)
        x_cps[1].wait()
        gemm(srows(c_dp1, 0), ccw_cols, ccw_cols)

        barrier = pltpu.get_barrier_semaphore()
        for nbr in (left, right):
            pl.semaphore_signal(barrier, inc=1, device_id=(nbr,),
                                device_id_type=pl.DeviceIdType.MESH)
        pl.semaphore_wait(barrier, 2)

        pend = {}
        pend[(0, 0)] = mk(0, 0, acc_ref.at[srows(c_dm1, 0), cw_cols],
                          acc_ref.at[srows(c_dp1, 0), ccw_cols],
                          cw_tgt=left, ccw_tgt=right)
        start(pend[(0, 0)])
        for j in range(1, SUBS):
            gemm(srows(c_dm1, j), cw_cols, cw_cols)
            gemm(srows(c_dp1, j), ccw_cols, ccw_cols)
            pend[(0, j)] = mk(0, j, acc_ref.at[srows(c_dm1, j), cw_cols],
                              acc_ref.at[srows(c_dp1, j), ccw_cols],
                              cw_tgt=left, ccw_tgt=right)
            start(pend[(0, j)])

        x_cps[2].wait()
        gemm(crows(c_dp2), cw_cols, cw_cols)
        gemm(crows(c_dp2), ccw_cols, ccw_cols)
        for j in range(SUBS):
            pend[(2, j)] = mk(2, j, acc_ref.at[srows(c_dp2, j), cw_cols],
                              acc_ref.at[srows(c_dp2, j), ccw_cols])
            start(pend[(2, j)])

        x_cps[3].wait()
        gemm(crows(d), cw_cols, cw_cols)
        gemm(crows(d), ccw_cols, ccw_cols)
        gemm(crows(oc_cw), cw_cols, cw_cols)
        gemm(crows(oc_ccw), ccw_cols, ccw_cols)

        for j in range(SUBS):
            wait(pend.pop((0, j)))
            acc_ref[srows(d, j), cw_cols] = (
                acc_ref[srows(d, j), cw_cols] + cw_ref[slot(0, j)])
            acc_ref[srows(d, j), ccw_cols] = (
                acc_ref[srows(d, j), ccw_cols] + ccw_ref[slot(0, j)])
            pend[(1, j)] = mk(1, j, acc_ref.at[srows(d, j), cw_cols],
                              acc_ref.at[srows(d, j), ccw_cols],
                              cw_tgt=left, ccw_tgt=right)
            start(pend[(1, j)])

        for j in range(SUBS):
            wait(pend.pop((1, j)))
            wait(pend.pop((2, j)))
            acc_ref[srows(oc_cw, j), cw_cols] = jnp.maximum(
                acc_ref[srows(oc_cw, j), cw_cols]
                + cw_ref[slot(1, j)] + cw_ref[slot(2, j)], 0.0)
            acc_ref[srows(oc_ccw, j), ccw_cols] = jnp.maximum(
                acc_ref[srows(oc_ccw, j), ccw_cols]
                + ccw_ref[slot(1, j)] + ccw_ref[slot(2, j)], 0.0)
            pend[(3, j)] = mk(3, j, acc_ref.at[srows(oc_cw, j), cw_cols],
                              acc_ref.at[srows(oc_ccw, j), ccw_cols])
            start(pend[(3, j)])
            pend[(4, j)] = mk(4, j, acc_ref.at[srows(oc_cw, j), cw_cols],
                              acc_ref.at[srows(oc_ccw, j), ccw_cols],
                              cw_tgt=left, ccw_tgt=right)
            start(pend[(4, j)])
            dma_out(acc_ref.at[srows(oc_cw, j), cw_cols],
                    out_ref.at[srows(oc_cw, j), cw_cols])
            dma_out(acc_ref.at[srows(oc_ccw, j), ccw_cols],
                    out_ref.at[srows(oc_ccw, j), ccw_cols])

        r_far = c_dp2
        r_fwd_cw = c_dm1
        r_fwd_ccw = c_dp1
        for j in range(SUBS):
            wait(pend.pop((3, j)))
            pend[(5, j)] = mk(5, j, cw_ref.at[slot(3, j)],
                              ccw_ref.at[slot(3, j)])
            start(pend[(5, j)])
            dma_out(cw_ref.at[slot(3, j)], out_ref.at[srows(d, j), cw_cols])
            dma_out(ccw_ref.at[slot(3, j)], out_ref.at[srows(d, j), ccw_cols])
        for j in range(SUBS):
            wait(pend.pop((4, j)))
            dma_out(cw_ref.at[slot(4, j)],
                    out_ref.at[srows(r_far, j), cw_cols])
            dma_out(ccw_ref.at[slot(4, j)],
                    out_ref.at[srows(r_far, j), ccw_cols])
        for j in range(SUBS):
            wait(pend.pop((5, j)))
            dma_out(cw_ref.at[slot(5, j)],
                    out_ref.at[srows(r_fwd_cw, j), cw_cols])
            dma_out(ccw_ref.at[slot(5, j)],
                    out_ref.at[srows(r_fwd_ccw, j), ccw_cols])

        for cp in out_dmas:
            cp.wait()

    return pl.pallas_call(
        body,
        out_shape=jax.ShapeDtypeStruct((m, n), jnp.float32),
        in_specs=[pl.BlockSpec(memory_space=pl.ANY),
                  pl.BlockSpec(memory_space=pl.ANY)],
        out_specs=pl.BlockSpec(memory_space=pl.ANY),
        scratch_shapes=[
            pltpu.VMEM((m, n), jnp.float32),
            pltpu.VMEM((m, k), jnp.float32),
            pltpu.VMEM((k, n), jnp.float32),
            pltpu.VMEM((n_slots, subrows, half), jnp.float32),
            pltpu.VMEM((n_slots, subrows, half), jnp.float32),
            pltpu.SemaphoreType.DMA((n_slots,)),
            pltpu.SemaphoreType.DMA((n_slots,)),
            pltpu.SemaphoreType.DMA((n_slots,)),
            pltpu.SemaphoreType.DMA((n_slots,)),
            pltpu.SemaphoreType.DMA((n_dmas,)),
            pltpu.SemaphoreType.DMA((5,)),
        ],
        compiler_params=pltpu.CompilerParams(
            collective_id=0, vmem_limit_bytes=100 * 1024 * 1024),
    )(x, w_mat)
